# baseline (speedup 1.0000x reference)
import functools

import jax
import jax.numpy as jnp
from jax import lax
from jax.experimental import pallas as pl
from jax.experimental.pallas import tpu as pltpu
from jax.experimental.pallas import tpu_sc as plsc

_L = 16


@functools.lru_cache(maxsize=None)
def _build_sc_kernel(B, S, D):
    info = plsc.get_sparse_core_info()
    NC, NS = info.num_cores, info.num_subcores
    NW = NC * NS
    PW = S // NW
    R = 8
    NCHUNK = PW // R
    assert S % NW == 0 and PW % R == 0 and D % _L == 0

    mesh = plsc.VectorSubcoreMesh(core_axis_name="c", subcore_axis_name="s")

    @functools.partial(
        pl.kernel,
        mesh=mesh,
        out_type=jax.ShapeDtypeStruct((B, S, D), jnp.float32),
        scratch_types=[
            pltpu.VMEM((3, B, R, D), jnp.float32),
            pltpu.VMEM((2, R, D), jnp.float32),
            pltpu.SemaphoreType.DMA((3,)),
            pltpu.SemaphoreType.DMA((3,)),
            pltpu.SemaphoreType.DMA((2,)),
        ],
    )
    def sc_kernel(x_hbm, pos_hbm, out_hbm, xb, pb, lsem, ssem, psem):
        wid = lax.axis_index("s") * NC + lax.axis_index("c")
        base = wid * PW

        def rows(g):
            return pl.ds(base + g * R, R)

        def start_load(g, buf):
            pltpu.async_copy(x_hbm.at[:, rows(g)], xb.at[buf], lsem.at[buf])

        def wait_load(buf):
            pltpu.make_async_copy(
                x_hbm.at[:, pl.ds(0, R)], xb.at[buf], lsem.at[buf]
            ).wait()

        def start_store(g, buf):
            pltpu.async_copy(xb.at[buf], out_hbm.at[:, rows(g)], ssem.at[buf])

        def wait_store(buf):
            pltpu.make_async_copy(
                xb.at[buf], out_hbm.at[:, pl.ds(0, R)], ssem.at[buf]
            ).wait()

        def start_pos(g, h):
            pltpu.async_copy(pos_hbm.at[rows(g)], pb.at[h], psem.at[h])

        def wait_pos(h):
            pltpu.make_async_copy(
                pos_hbm.at[pl.ds(0, R)], pb.at[h], psem.at[h]
            ).wait()

        CPR = D // _L
        RC = R * CPR

        def compute(buf, h):
            @plsc.parallel_loop(0, B * RC, unroll=8)
            def _(i):
                b = i // RC
                r = (i // CPR) % R
                sl = pl.ds((i % CPR) * _L, _L)
                plsc.addupdate(xb.at[buf].at[b].at[r].at[sl], pb[h, r, sl])

        start_pos(0, 0)
        start_pos(1, 1)
        start_load(0, 0)

        for g in range(NCHUNK):
            if g + 1 < NCHUNK:
                if g >= 2:
                    wait_store((g + 1) % 3)
                start_load(g + 1, (g + 1) % 3)
            wait_pos(g % 2)
            wait_load(g % 3)
            compute(g % 3, g % 2)
            start_store(g, g % 3)
            if g + 2 < NCHUNK:
                start_pos(g + 2, g % 2)

        for g in range(NCHUNK - 3, NCHUNK):
            wait_store(g % 3)

    return sc_kernel


def kernel(x, pos_table):
    B, S, D = x.shape
    sc = _build_sc_kernel(B, S, D)
    return sc(x, pos_table[:S])


# speedup vs baseline: 1.0545x; 1.0545x over previous
import functools

import jax
import jax.numpy as jnp
from jax import lax
from jax.experimental import pallas as pl
from jax.experimental.pallas import tpu as pltpu
from jax.experimental.pallas import tpu_sc as plsc

_L = 16


@functools.lru_cache(maxsize=None)
def _build_sc_kernel(B, S, D):
    info = plsc.get_sparse_core_info()
    NC, NS = info.num_cores, info.num_subcores
    NW = NC * NS
    PW = S // NW
    R = 8
    NCHUNK = PW // R
    assert S % NW == 0 and PW % R == 0 and D % _L == 0
    assert B == 4 and NCHUNK % 2 == 0

    mesh = plsc.VectorSubcoreMesh(core_axis_name="c", subcore_axis_name="s")

    @functools.partial(
        pl.kernel,
        mesh=mesh,
        out_type=jax.ShapeDtypeStruct((B, S, D), jnp.float32),
        scratch_types=[
            pltpu.VMEM((8, R, D), jnp.float32),
            pltpu.VMEM((2, R, D), jnp.float32),
            pltpu.SemaphoreType.DMA((8,)),
            pltpu.SemaphoreType.DMA((8,)),
            pltpu.SemaphoreType.DMA((2,)),
        ],
    )
    def sc_kernel(x_hbm, pos_hbm, out_hbm, xb, pb, lsem, ssem, psem):
        wid = lax.axis_index("s") * NC + lax.axis_index("c")
        base = wid * PW

        def rows(g):
            return pl.ds(base + g * R, R)

        def start_load(g, j, buf):
            pltpu.async_copy(x_hbm.at[j, rows(g)], xb.at[buf], lsem.at[buf])

        def wait_load(buf):
            pltpu.make_async_copy(
                x_hbm.at[0, pl.ds(0, R)], xb.at[buf], lsem.at[buf]
            ).wait()

        def start_store(g, j, buf):
            pltpu.async_copy(xb.at[buf], out_hbm.at[j, rows(g)], ssem.at[buf])

        def wait_store(buf):
            pltpu.make_async_copy(
                xb.at[buf], out_hbm.at[0, pl.ds(0, R)], ssem.at[buf]
            ).wait()

        def start_pos(g, h):
            pltpu.async_copy(pos_hbm.at[rows(g)], pb.at[h], psem.at[h])

        def wait_pos(h):
            pltpu.make_async_copy(
                pos_hbm.at[pl.ds(0, R)], pb.at[h], psem.at[h]
            ).wait()

        CPR = D // _L

        def compute(buf, h):
            @plsc.parallel_loop(0, R * CPR, unroll=8)
            def _(i):
                r = i // CPR
                sl = pl.ds((i % CPR) * _L, _L)
                plsc.addupdate(xb.at[buf].at[r].at[sl], pb[h, r, sl])

        start_pos(0, 0)
        start_pos(1, 1)
        for j in range(4):
            start_load(0, j, j)

        @pl.loop(0, NCHUNK // 2)
        def _(gg):
            for h in range(2):
                g = 2 * gg + h
                wait_pos(h)
                for j in range(4):
                    beta = 4 * h + j
                    bn = 4 * (1 - h) + j
                    @pl.when(g < NCHUNK - 1)
                    def _():
                        @pl.when(g >= 1)
                        def _():
                            wait_store(bn)
                        start_load(g + 1, j, bn)
                    wait_load(beta)
                    compute(beta, h)
                    start_store(g, j, beta)
                @pl.when(g < NCHUNK - 2)
                def _():
                    start_pos(g + 2, h)

        for buf in range(8):
            wait_store(buf)

    return sc_kernel


def kernel(x, pos_table):
    B, S, D = x.shape
    sc = _build_sc_kernel(B, S, D)
    return sc(x, pos_table[:S])
